# baseline (speedup 1.0000x reference)
import functools

import jax
import jax.numpy as jnp
from jax import lax
from jax.experimental import pallas as pl
from jax.experimental.pallas import tpu as pltpu
from jax.experimental.pallas import tpu_sc as plsc

N = 10000
E = 320000
D = 128

NC = 2
NS = 16
NW = NC * NS

CH = 128
EPT = 10240
NCHUNK = EPT // CH
E_PAD = EPT * NW
NA = 10240
STRIPE = NA // NS
DUMMY = N

G = 8
NGRP = NCHUNK // G

CHD = 128
NCHD = EPT // CHD
GD = 8
NGRPD = NCHD // GD


@functools.cache
def _make_deg_kernel():
    mesh = plsc.VectorSubcoreMesh(core_axis_name="c", subcore_axis_name="s")
    return functools.partial(
        pl.kernel,
        mesh=mesh,
        out_type=jax.ShapeDtypeStruct((NC * NA, 16), jnp.float32),
        scratch_types=[
            pltpu.VMEM_SHARED((NA, 16), jnp.float32),
            pltpu.VMEM((GD, CHD), jnp.int32),
            pltpu.VMEM((CHD, 16), jnp.float32),
        ],
    )(_deg_body)


def _deg_body(dst_hbm, out_hbm, acc, dst_v, ones_v):
    c = lax.axis_index("c")
    s = lax.axis_index("s")
    wid = c * NS + s

    zero16 = jnp.zeros((1, 16), jnp.float32)
    one16 = jnp.ones((1, 16), jnp.float32)

    @pl.loop(0, CHD)
    def _(i):
        ones_v.at[pl.ds(i, 1), pl.ds(0, 16)][...] = zero16

    @pl.loop(0, STRIPE // CHD)
    def _(t):
        pltpu.sync_copy(ones_v, acc.at[pl.ds(s * STRIPE + t * CHD, CHD)])

    @pl.loop(0, CHD)
    def _(i):
        ones_v.at[pl.ds(i, 1), pl.ds(0, 16)][...] = one16

    plsc.subcore_barrier()

    @pl.loop(0, NGRPD)
    def _(g):
        pltpu.sync_copy(dst_hbm.at[wid, pl.ds(g * GD, GD)], dst_v)

        @pl.loop(0, GD)
        def _(i):
            pltpu.sync_copy(ones_v, acc.at[dst_v.at[i]], add=True)

    plsc.subcore_barrier()
    pltpu.sync_copy(
        acc.at[pl.ds(s * STRIPE, STRIPE)],
        out_hbm.at[pl.ds(c * NA + s * STRIPE, STRIPE)],
    )


@functools.cache
def _make_scatter_kernel():
    mesh = plsc.VectorSubcoreMesh(core_axis_name="c", subcore_axis_name="s")
    return functools.partial(
        pl.kernel,
        mesh=mesh,
        out_type=jax.ShapeDtypeStruct((NC * NA, D), jnp.float32),
        scratch_types=[
            pltpu.VMEM_SHARED((NA, D), jnp.float32),
            pltpu.VMEM((G, CH), jnp.int32),
            pltpu.VMEM((G, CH), jnp.int32),
            pltpu.VMEM((CH, D), jnp.float32),
            pltpu.VMEM((CH, D), jnp.float32),
            pltpu.SemaphoreType.DMA,
            pltpu.SemaphoreType.DMA,
            pltpu.SemaphoreType.DMA,
            pltpu.SemaphoreType.DMA,
        ],
    )(_scatter_body)


def _scatter_body(g_hbm, src_hbm, dst_hbm, out_hbm, acc, src_v, dst_v,
                  r0, r1, sg0, sg1, ss0, ss1):
    c = lax.axis_index("c")
    s = lax.axis_index("s")
    wid = c * NS + s

    zero16 = jnp.zeros((1, 16), jnp.float32)

    @pl.loop(0, CH)
    def _(i):
        @pl.loop(0, D, step=16)
        def _(j):
            r0.at[pl.ds(i, 1), pl.ds(j, 16)][...] = zero16

    @pl.loop(0, STRIPE // CH)
    def _(t):
        pltpu.sync_copy(r0, acc.at[pl.ds(s * STRIPE + t * CH, CH)])

    plsc.subcore_barrier()

    bufs = ((r0, sg0, ss0), (r1, sg1, ss1))

    @pl.loop(0, NGRP)
    def _(grp):
        pltpu.sync_copy(src_hbm.at[wid, pl.ds(grp * G, G)], src_v)
        pltpu.sync_copy(dst_hbm.at[wid, pl.ds(grp * G, G)], dst_v)

        for b, (rb, gsem, _ssem) in enumerate(bufs):
            pltpu.async_copy(g_hbm.at[src_v.at[b]], rb, gsem)

        @pl.loop(0, G - 2, step=2)
        def _(i):
            for b, (rb, gsem, ssem) in enumerate(bufs):
                pltpu.make_async_copy(g_hbm.at[src_v.at[i + b]], rb, gsem).wait()
                pltpu.async_copy(rb, acc.at[dst_v.at[i + b]], ssem, add=True)
            for b, (rb, gsem, ssem) in enumerate(bufs):
                pltpu.make_async_copy(rb, acc.at[dst_v.at[i + b]], ssem).wait()
                pltpu.async_copy(g_hbm.at[src_v.at[i + b + 2]], rb, gsem)

        for b, (rb, gsem, ssem) in enumerate(bufs):
            i = G - 2 + b
            pltpu.make_async_copy(g_hbm.at[src_v.at[i]], rb, gsem).wait()
            pltpu.async_copy(rb, acc.at[dst_v.at[i]], ssem, add=True)
        for b, (rb, gsem, ssem) in enumerate(bufs):
            i = G - 2 + b
            pltpu.make_async_copy(rb, acc.at[dst_v.at[i]], ssem).wait()

    plsc.subcore_barrier()
    pltpu.sync_copy(
        acc.at[pl.ds(s * STRIPE, STRIPE)],
        out_hbm.at[pl.ds(c * NA + s * STRIPE, STRIPE)],
    )


_PREC = lax.Precision.HIGHEST
BLK = 1000
GRID = N // BLK


def _dinv_of(p_ref):
    deg = 1.0 + p_ref[0, :, 0:1] + p_ref[1, :, 0:1]
    return lax.rsqrt(deg)


_P16 = 16


def _tc_matmul_scale(p, x, w):
    def body(p_ref, x_ref, w_ref, o_ref):
        h = jnp.dot(x_ref[...], w_ref[...],
                    preferred_element_type=jnp.float32,
                    precision=_PREC)
        o_ref[...] = h * _dinv_of(p_ref)

    return pl.pallas_call(
        body,
        out_shape=jax.ShapeDtypeStruct((N, D), jnp.float32),
        grid=(GRID,),
        in_specs=[
            pl.BlockSpec((2, BLK, _P16), lambda i: (0, i, 0)),
            pl.BlockSpec((BLK, D), lambda i: (i, 0)),
            pl.BlockSpec((D, D), lambda i: (0, 0)),
        ],
        out_specs=pl.BlockSpec((BLK, D), lambda i: (i, 0)),
    )(p, x, w)


def _tc_scale(p, h):
    def body(p_ref, h_ref, o_ref):
        o_ref[...] = h_ref[...] * _dinv_of(p_ref)

    return pl.pallas_call(
        body,
        out_shape=jax.ShapeDtypeStruct((N, D), jnp.float32),
        grid=(GRID,),
        in_specs=[
            pl.BlockSpec((2, BLK, _P16), lambda i: (0, i, 0)),
            pl.BlockSpec((BLK, D), lambda i: (i, 0)),
        ],
        out_specs=pl.BlockSpec((BLK, D), lambda i: (i, 0)),
    )(p, h)


def _tc_layer2(p, s1, g1, b1, w2):
    def body(p_ref, s_ref, g_ref, b_ref, w_ref, o_ref):
        dinv = _dinv_of(p_ref)
        y = (s_ref[0] + s_ref[1] + g_ref[...]) * dinv + b_ref[...]
        y = jnp.maximum(y, 0.0)
        h2 = jnp.dot(y, w_ref[...], preferred_element_type=jnp.float32,
                     precision=_PREC)
        o_ref[...] = h2 * dinv

    return pl.pallas_call(
        body,
        out_shape=jax.ShapeDtypeStruct((N, D), jnp.float32),
        grid=(GRID,),
        in_specs=[
            pl.BlockSpec((2, BLK, _P16), lambda i: (0, i, 0)),
            pl.BlockSpec((2, BLK, D), lambda i: (0, i, 0)),
            pl.BlockSpec((BLK, D), lambda i: (i, 0)),
            pl.BlockSpec((1, D), lambda i: (0, 0)),
            pl.BlockSpec((D, D), lambda i: (0, 0)),
        ],
        out_specs=pl.BlockSpec((BLK, D), lambda i: (i, 0)),
    )(p, s1, g1, b1, w2)


def _tc_out(p, s2, g2, b2):
    def body(p_ref, s_ref, g_ref, b_ref, o_ref):
        dinv = _dinv_of(p_ref)
        o_ref[...] = (s_ref[0] + s_ref[1] + g_ref[...]) * dinv + b_ref[...]

    return pl.pallas_call(
        body,
        out_shape=jax.ShapeDtypeStruct((N, D), jnp.float32),
        grid=(GRID,),
        in_specs=[
            pl.BlockSpec((2, BLK, _P16), lambda i: (0, i, 0)),
            pl.BlockSpec((2, BLK, D), lambda i: (0, i, 0)),
            pl.BlockSpec((BLK, D), lambda i: (i, 0)),
            pl.BlockSpec((1, D), lambda i: (0, 0)),
        ],
        out_specs=pl.BlockSpec((BLK, D), lambda i: (i, 0)),
    )(p, s2, g2, b2)


def kernel(x, edge_index, W1, b1, W2, b2):
    src = edge_index[0]
    dst = edge_index[1]
    pad = E_PAD - E
    pad_src = (jnp.arange(pad, dtype=jnp.int32) * 131) % N
    pad_dst = N + (jnp.arange(pad, dtype=jnp.int32) % (NA - N))
    src_p = jnp.concatenate([src, pad_src]).reshape(NW, NCHUNK, CH)
    dst_p = jnp.concatenate([dst, pad_dst]).reshape(NW, NCHUNK, CH)

    scatter_kernel = _make_scatter_kernel()
    deg_kernel = _make_deg_kernel()
    dst_d = dst_p.reshape(NW, NCHD, CHD)
    p = deg_kernel(dst_d).reshape(NC, NA, 16)
    g1 = _tc_matmul_scale(p, x, W1)
    s1 = scatter_kernel(g1, src_p, dst_p).reshape(NC, NA, D)
    g2 = _tc_layer2(p, s1, g1, b1.reshape(1, D), W2)
    s2 = scatter_kernel(g2, src_p, dst_p).reshape(NC, NA, D)
    return _tc_out(p, s2, g2, b2.reshape(1, D))


# speedup vs baseline: 1.0088x; 1.0088x over previous
import functools

import jax
import jax.numpy as jnp
from jax import lax
from jax.experimental import pallas as pl
from jax.experimental.pallas import tpu as pltpu
from jax.experimental.pallas import tpu_sc as plsc

N = 10000
E = 320000
D = 128

NC = 2
NS = 16
NW = NC * NS

CH = 128
EPT = 10240
NCHUNK = EPT // CH
E_PAD = EPT * NW
NA = 10240
STRIPE = NA // NS
DUMMY = N

G = 8
NGRP = NCHUNK // G

CHD = 128
NCHD = EPT // CHD
GD = 8
NGRPD = NCHD // GD


@functools.cache
def _make_deg_kernel():
    mesh = plsc.VectorSubcoreMesh(core_axis_name="c", subcore_axis_name="s")
    return functools.partial(
        pl.kernel,
        mesh=mesh,
        out_type=jax.ShapeDtypeStruct((NC * NA, 16), jnp.float32),
        scratch_types=[
            pltpu.VMEM_SHARED((NA, 16), jnp.float32),
            pltpu.VMEM((GD, CHD), jnp.int32),
            pltpu.VMEM((CHD, 16), jnp.float32),
        ],
    )(_deg_body)


def _deg_body(dst_hbm, out_hbm, acc, dst_v, ones_v):
    c = lax.axis_index("c")
    s = lax.axis_index("s")
    wid = c * NS + s

    zero16 = jnp.zeros((1, 16), jnp.float32)
    one16 = jnp.ones((1, 16), jnp.float32)

    @pl.loop(0, CHD)
    def _(i):
        ones_v.at[pl.ds(i, 1), pl.ds(0, 16)][...] = zero16

    @pl.loop(0, STRIPE // CHD)
    def _(t):
        pltpu.sync_copy(ones_v, acc.at[pl.ds(s * STRIPE + t * CHD, CHD)])

    @pl.loop(0, CHD)
    def _(i):
        ones_v.at[pl.ds(i, 1), pl.ds(0, 16)][...] = one16

    plsc.subcore_barrier()

    @pl.loop(0, NGRPD)
    def _(g):
        pltpu.sync_copy(dst_hbm.at[wid, pl.ds(g * GD, GD)], dst_v)

        @pl.loop(0, GD)
        def _(i):
            pltpu.sync_copy(ones_v, acc.at[dst_v.at[i]], add=True)

    plsc.subcore_barrier()
    pltpu.sync_copy(
        acc.at[pl.ds(s * STRIPE, STRIPE)],
        out_hbm.at[pl.ds(c * NA + s * STRIPE, STRIPE)],
    )


@functools.cache
def _make_scatter_kernel():
    mesh = plsc.VectorSubcoreMesh(core_axis_name="c", subcore_axis_name="s")
    return functools.partial(
        pl.kernel,
        mesh=mesh,
        out_type=jax.ShapeDtypeStruct((NC * NA, D), jnp.float32),
        scratch_types=[
            pltpu.VMEM_SHARED((NA, D), jnp.float32),
            pltpu.VMEM((G, CH), jnp.int32),
            pltpu.VMEM((G, CH), jnp.int32),
            pltpu.VMEM((CH, D), jnp.float32),
            pltpu.VMEM((CH, D), jnp.float32),
            pltpu.SemaphoreType.DMA,
            pltpu.SemaphoreType.DMA,
            pltpu.SemaphoreType.DMA,
            pltpu.SemaphoreType.DMA,
        ],
    )(_scatter_body)


def _scatter_body(g_hbm, src_hbm, dst_hbm, out_hbm, acc, src_v, dst_v,
                  r0, r1, sg0, sg1, ss0, ss1):
    c = lax.axis_index("c")
    s = lax.axis_index("s")
    wid = c * NS + s

    zero16 = jnp.zeros((1, 16), jnp.float32)

    @pl.loop(0, CH)
    def _(i):
        @pl.loop(0, D, step=16)
        def _(j):
            r0.at[pl.ds(i, 1), pl.ds(j, 16)][...] = zero16

    @pl.loop(0, STRIPE // CH)
    def _(t):
        pltpu.sync_copy(r0, acc.at[pl.ds(s * STRIPE + t * CH, CH)])

    plsc.subcore_barrier()

    bufs = ((r0, sg0, ss0), (r1, sg1, ss1))

    @pl.loop(0, NGRP)
    def _(grp):
        pltpu.sync_copy(src_hbm.at[wid, pl.ds(grp * G, G)], src_v)
        pltpu.sync_copy(dst_hbm.at[wid, pl.ds(grp * G, G)], dst_v)

        for b, (rb, gsem, _ssem) in enumerate(bufs):
            pltpu.async_copy(g_hbm.at[src_v.at[b]], rb, gsem)

        @pl.loop(0, G - 2, step=2)
        def _(i):
            for b, (rb, gsem, ssem) in enumerate(bufs):
                pltpu.make_async_copy(g_hbm.at[src_v.at[i + b]], rb, gsem).wait()
                pltpu.async_copy(rb, acc.at[dst_v.at[i + b]], ssem, add=True)
            for b, (rb, gsem, ssem) in enumerate(bufs):
                pltpu.make_async_copy(rb, acc.at[dst_v.at[i + b]], ssem).wait()
                pltpu.async_copy(g_hbm.at[src_v.at[i + b + 2]], rb, gsem)

        for b, (rb, gsem, ssem) in enumerate(bufs):
            i = G - 2 + b
            pltpu.make_async_copy(g_hbm.at[src_v.at[i]], rb, gsem).wait()
            pltpu.async_copy(rb, acc.at[dst_v.at[i]], ssem, add=True)
        for b, (rb, gsem, ssem) in enumerate(bufs):
            i = G - 2 + b
            pltpu.make_async_copy(rb, acc.at[dst_v.at[i]], ssem).wait()

    plsc.subcore_barrier()
    pltpu.sync_copy(
        acc.at[pl.ds(s * STRIPE, STRIPE)],
        out_hbm.at[pl.ds(c * NA + s * STRIPE, STRIPE)],
    )


_PREC = lax.Precision.HIGHEST
BLK = 1000
GRID = N // BLK


def _dinv_of(p_ref):
    deg = 1.0 + p_ref[0, :, 0:1] + p_ref[1, :, 0:1]
    return lax.rsqrt(deg)


_P16 = 16


def _tc_matmul(x, w):
    def body(x_ref, w_ref, o_ref):
        o_ref[...] = jnp.dot(x_ref[...], w_ref[...],
                             preferred_element_type=jnp.float32,
                             precision=_PREC)

    return pl.pallas_call(
        body,
        out_shape=jax.ShapeDtypeStruct((N, D), jnp.float32),
        grid=(GRID,),
        in_specs=[
            pl.BlockSpec((BLK, D), lambda i: (i, 0)),
            pl.BlockSpec((D, D), lambda i: (0, 0)),
        ],
        out_specs=pl.BlockSpec((BLK, D), lambda i: (i, 0)),
    )(x, w)


def _tc_scale(p, h):
    def body(p_ref, h_ref, o_ref):
        o_ref[...] = h_ref[...] * _dinv_of(p_ref)

    return pl.pallas_call(
        body,
        out_shape=jax.ShapeDtypeStruct((N, D), jnp.float32),
        grid=(GRID,),
        in_specs=[
            pl.BlockSpec((2, BLK, _P16), lambda i: (0, i, 0)),
            pl.BlockSpec((BLK, D), lambda i: (i, 0)),
        ],
        out_specs=pl.BlockSpec((BLK, D), lambda i: (i, 0)),
    )(p, h)


def _tc_layer2(p, s1, g1, b1, w2):
    def body(p_ref, s_ref, g_ref, b_ref, w_ref, o_ref):
        dinv = _dinv_of(p_ref)
        y = (s_ref[0] + s_ref[1] + g_ref[...]) * dinv + b_ref[...]
        y = jnp.maximum(y, 0.0)
        h2 = jnp.dot(y, w_ref[...], preferred_element_type=jnp.float32,
                     precision=_PREC)
        o_ref[...] = h2 * dinv

    return pl.pallas_call(
        body,
        out_shape=jax.ShapeDtypeStruct((N, D), jnp.float32),
        grid=(GRID,),
        in_specs=[
            pl.BlockSpec((2, BLK, _P16), lambda i: (0, i, 0)),
            pl.BlockSpec((2, BLK, D), lambda i: (0, i, 0)),
            pl.BlockSpec((BLK, D), lambda i: (i, 0)),
            pl.BlockSpec((1, D), lambda i: (0, 0)),
            pl.BlockSpec((D, D), lambda i: (0, 0)),
        ],
        out_specs=pl.BlockSpec((BLK, D), lambda i: (i, 0)),
    )(p, s1, g1, b1, w2)


def _tc_out(p, s2, g2, b2):
    def body(p_ref, s_ref, g_ref, b_ref, o_ref):
        dinv = _dinv_of(p_ref)
        o_ref[...] = (s_ref[0] + s_ref[1] + g_ref[...]) * dinv + b_ref[...]

    return pl.pallas_call(
        body,
        out_shape=jax.ShapeDtypeStruct((N, D), jnp.float32),
        grid=(GRID,),
        in_specs=[
            pl.BlockSpec((2, BLK, _P16), lambda i: (0, i, 0)),
            pl.BlockSpec((2, BLK, D), lambda i: (0, i, 0)),
            pl.BlockSpec((BLK, D), lambda i: (i, 0)),
            pl.BlockSpec((1, D), lambda i: (0, 0)),
        ],
        out_specs=pl.BlockSpec((BLK, D), lambda i: (i, 0)),
    )(p, s2, g2, b2)


def kernel(x, edge_index, W1, b1, W2, b2):
    src = edge_index[0]
    dst = edge_index[1]
    pad = E_PAD - E
    pad_src = (jnp.arange(pad, dtype=jnp.int32) * 131) % N
    pad_dst = N + (jnp.arange(pad, dtype=jnp.int32) % (NA - N))
    src_p = jnp.concatenate([src, pad_src]).reshape(NW, NCHUNK, CH)
    dst_p = jnp.concatenate([dst, pad_dst]).reshape(NW, NCHUNK, CH)

    scatter_kernel = _make_scatter_kernel()
    deg_kernel = _make_deg_kernel()
    dst_d = dst_p.reshape(NW, NCHD, CHD)
    p = deg_kernel(dst_d).reshape(NC, NA, 16)
    h1 = _tc_matmul(x, W1)
    g1 = _tc_scale(p, h1)
    s1 = scatter_kernel(g1, src_p, dst_p).reshape(NC, NA, D)
    g2 = _tc_layer2(p, s1, g1, b1.reshape(1, D), W2)
    s2 = scatter_kernel(g2, src_p, dst_p).reshape(NC, NA, D)
    return _tc_out(p, s2, g2, b2.reshape(1, D))


# speedup vs baseline: 1.1568x; 1.1468x over previous
import functools

import jax
import jax.numpy as jnp
from jax import lax
from jax.experimental import pallas as pl
from jax.experimental.pallas import tpu as pltpu
from jax.experimental.pallas import tpu_sc as plsc

N = 10000
E = 320000
D = 128

NC = 2
NS = 16
NW = NC * NS

CH = 128
EPT = 10240
NCHUNK = EPT // CH
E_PAD = EPT * NW
NA = 10240
STRIPE = NA // NS
DUMMY = N

G = 8
NGRP = NCHUNK // G

CHD = 128
NCHD = EPT // CHD
GD = 8
NGRPD = NCHD // GD


@functools.cache
def _make_deg_kernel():
    mesh = plsc.VectorSubcoreMesh(core_axis_name="c", subcore_axis_name="s")
    return functools.partial(
        pl.kernel,
        mesh=mesh,
        out_type=jax.ShapeDtypeStruct((NC * NA, 16), jnp.float32),
        scratch_types=[
            pltpu.VMEM_SHARED((NA, 16), jnp.float32),
            pltpu.VMEM((GD, CHD), jnp.int32),
            pltpu.VMEM((CHD, 16), jnp.float32),
        ],
    )(_deg_body)


def _deg_body(dst_hbm, out_hbm, acc, dst_v, ones_v):
    c = lax.axis_index("c")
    s = lax.axis_index("s")
    wid = c * NS + s

    zero16 = jnp.zeros((1, 16), jnp.float32)
    one16 = jnp.ones((1, 16), jnp.float32)

    @pl.loop(0, CHD)
    def _(i):
        ones_v.at[pl.ds(i, 1), pl.ds(0, 16)][...] = zero16

    @pl.loop(0, STRIPE // CHD)
    def _(t):
        pltpu.sync_copy(ones_v, acc.at[pl.ds(s * STRIPE + t * CHD, CHD)])

    @pl.loop(0, CHD)
    def _(i):
        ones_v.at[pl.ds(i, 1), pl.ds(0, 16)][...] = one16

    plsc.subcore_barrier()

    @pl.loop(0, NGRPD)
    def _(g):
        pltpu.sync_copy(dst_hbm.at[wid, pl.ds(g * GD, GD)], dst_v)

        @pl.loop(0, GD)
        def _(i):
            pltpu.sync_copy(ones_v, acc.at[dst_v.at[i]], add=True)

    plsc.subcore_barrier()
    pltpu.sync_copy(
        acc.at[pl.ds(s * STRIPE, STRIPE)],
        out_hbm.at[pl.ds(c * NA + s * STRIPE, STRIPE)],
    )


@functools.cache
def _make_scatter_kernel():
    mesh = plsc.VectorSubcoreMesh(core_axis_name="c", subcore_axis_name="s")
    return functools.partial(
        pl.kernel,
        mesh=mesh,
        out_type=jax.ShapeDtypeStruct((NC * NA, D), jnp.float32),
        scratch_types=[
            pltpu.VMEM_SHARED((NA, D), jnp.float32),
            pltpu.VMEM((G, CH), jnp.int32),
            pltpu.VMEM((G, CH), jnp.int32),
            pltpu.VMEM((CH, D), jnp.float32),
            pltpu.VMEM((CH, D), jnp.float32),
            pltpu.SemaphoreType.DMA,
            pltpu.SemaphoreType.DMA,
        ],
    )(_scatter_body)


def _scatter_body(g_hbm, src_hbm, dst_hbm, out_hbm, acc, src_v, dst_v,
                  r0, r1, sg0, sg1):
    c = lax.axis_index("c")
    s = lax.axis_index("s")
    wid = c * NS + s

    zero16 = jnp.zeros((1, 16), jnp.float32)

    @pl.loop(0, CH)
    def _(i):
        @pl.loop(0, D, step=16)
        def _(j):
            r0.at[pl.ds(i, 1), pl.ds(j, 16)][...] = zero16

    @pl.loop(0, STRIPE // CH)
    def _(t):
        pltpu.sync_copy(r0, acc.at[pl.ds(s * STRIPE + t * CH, CH)])

    plsc.subcore_barrier()

    bufs = ((r0, sg0), (r1, sg1))

    @pl.loop(0, NGRP)
    def _(grp):
        pltpu.sync_copy(src_hbm.at[wid, pl.ds(grp * G, G)], src_v)
        pltpu.sync_copy(dst_hbm.at[wid, pl.ds(grp * G, G)], dst_v)

        for b, (rb, gsem) in enumerate(bufs):
            pltpu.async_copy(g_hbm.at[src_v.at[b]], rb, gsem)

        @pl.loop(0, G - 2, step=2)
        def _(i):
            for b, (rb, gsem) in enumerate(bufs):
                pltpu.make_async_copy(g_hbm.at[src_v.at[i + b]], rb, gsem).wait()
                pltpu.sync_copy(rb, acc.at[dst_v.at[i + b]], add=True)
                pltpu.async_copy(g_hbm.at[src_v.at[i + b + 2]], rb, gsem)

        for b, (rb, gsem) in enumerate(bufs):
            i = G - 2 + b
            pltpu.make_async_copy(g_hbm.at[src_v.at[i]], rb, gsem).wait()
            pltpu.sync_copy(rb, acc.at[dst_v.at[i]], add=True)

    plsc.subcore_barrier()
    pltpu.sync_copy(
        acc.at[pl.ds(s * STRIPE, STRIPE)],
        out_hbm.at[pl.ds(c * NA + s * STRIPE, STRIPE)],
    )


_PREC = lax.Precision.HIGHEST
BLK = 1000
GRID = N // BLK


def _dinv_of(p_ref):
    deg = 1.0 + p_ref[0, :, 0:1] + p_ref[1, :, 0:1]
    return lax.rsqrt(deg)


_P16 = 16


def _tc_matmul(x, w):
    def body(x_ref, w_ref, o_ref):
        o_ref[...] = jnp.dot(x_ref[...], w_ref[...],
                             preferred_element_type=jnp.float32,
                             precision=_PREC)

    return pl.pallas_call(
        body,
        out_shape=jax.ShapeDtypeStruct((N, D), jnp.float32),
        grid=(GRID,),
        in_specs=[
            pl.BlockSpec((BLK, D), lambda i: (i, 0)),
            pl.BlockSpec((D, D), lambda i: (0, 0)),
        ],
        out_specs=pl.BlockSpec((BLK, D), lambda i: (i, 0)),
    )(x, w)


def _tc_scale(p, h):
    def body(p_ref, h_ref, o_ref):
        o_ref[...] = h_ref[...] * _dinv_of(p_ref)

    return pl.pallas_call(
        body,
        out_shape=jax.ShapeDtypeStruct((N, D), jnp.float32),
        grid=(GRID,),
        in_specs=[
            pl.BlockSpec((2, BLK, _P16), lambda i: (0, i, 0)),
            pl.BlockSpec((BLK, D), lambda i: (i, 0)),
        ],
        out_specs=pl.BlockSpec((BLK, D), lambda i: (i, 0)),
    )(p, h)


def _tc_layer2(p, s1, g1, b1, w2):
    def body(p_ref, s_ref, g_ref, b_ref, w_ref, o_ref):
        dinv = _dinv_of(p_ref)
        y = (s_ref[0] + s_ref[1] + g_ref[...]) * dinv + b_ref[...]
        y = jnp.maximum(y, 0.0)
        h2 = jnp.dot(y, w_ref[...], preferred_element_type=jnp.float32,
                     precision=_PREC)
        o_ref[...] = h2 * dinv

    return pl.pallas_call(
        body,
        out_shape=jax.ShapeDtypeStruct((N, D), jnp.float32),
        grid=(GRID,),
        in_specs=[
            pl.BlockSpec((2, BLK, _P16), lambda i: (0, i, 0)),
            pl.BlockSpec((2, BLK, D), lambda i: (0, i, 0)),
            pl.BlockSpec((BLK, D), lambda i: (i, 0)),
            pl.BlockSpec((1, D), lambda i: (0, 0)),
            pl.BlockSpec((D, D), lambda i: (0, 0)),
        ],
        out_specs=pl.BlockSpec((BLK, D), lambda i: (i, 0)),
    )(p, s1, g1, b1, w2)


def _tc_out(p, s2, g2, b2):
    def body(p_ref, s_ref, g_ref, b_ref, o_ref):
        dinv = _dinv_of(p_ref)
        o_ref[...] = (s_ref[0] + s_ref[1] + g_ref[...]) * dinv + b_ref[...]

    return pl.pallas_call(
        body,
        out_shape=jax.ShapeDtypeStruct((N, D), jnp.float32),
        grid=(GRID,),
        in_specs=[
            pl.BlockSpec((2, BLK, _P16), lambda i: (0, i, 0)),
            pl.BlockSpec((2, BLK, D), lambda i: (0, i, 0)),
            pl.BlockSpec((BLK, D), lambda i: (i, 0)),
            pl.BlockSpec((1, D), lambda i: (0, 0)),
        ],
        out_specs=pl.BlockSpec((BLK, D), lambda i: (i, 0)),
    )(p, s2, g2, b2)


def kernel(x, edge_index, W1, b1, W2, b2):
    src = edge_index[0]
    dst = edge_index[1]
    pad = E_PAD - E
    pad_src = (jnp.arange(pad, dtype=jnp.int32) * 131) % N
    pad_dst = N + (jnp.arange(pad, dtype=jnp.int32) % (NA - N))
    src_p = jnp.concatenate([src, pad_src]).reshape(NW, NCHUNK, CH)
    dst_p = jnp.concatenate([dst, pad_dst]).reshape(NW, NCHUNK, CH)

    scatter_kernel = _make_scatter_kernel()
    deg_kernel = _make_deg_kernel()
    dst_d = dst_p.reshape(NW, NCHD, CHD)
    p = deg_kernel(dst_d).reshape(NC, NA, 16)
    h1 = _tc_matmul(x, W1)
    g1 = _tc_scale(p, h1)
    s1 = scatter_kernel(g1, src_p, dst_p).reshape(NC, NA, D)
    g2 = _tc_layer2(p, s1, g1, b1.reshape(1, D), W2)
    s2 = scatter_kernel(g2, src_p, dst_p).reshape(NC, NA, D)
    return _tc_out(p, s2, g2, b2.reshape(1, D))
